# baseline (speedup 1.0000x reference)
import functools
import math

import jax
import jax.numpy as jnp
from jax import lax
from jax.experimental import pallas as pl
from jax.experimental.pallas import tpu as pltpu, tpu_sc as plsc

NC = 2
NS = 16
NW = NC * NS
CHUNK = 128
KB1 = 8
KBT = 12


def _round1_body(nb, cpt, rpt,
                 x_hbm, sd_hbm, ea_hbm, z8_hbm,
                 outx_hbm, oute_hbm,
                 sdb, rows, eab, accx_s, acce_s, sem, sem2):
  c = lax.axis_index("c")
  s = lax.axis_index("s")
  w = c * NS + s
  pltpu.sync_copy(z8_hbm, accx_s.at[pl.ds(s * rpt, rpt)])
  pltpu.sync_copy(z8_hbm, acce_s.at[pl.ds(s * rpt, rpt)])
  plsc.subcore_barrier()

  def block(i, carry):
    row0 = w * cpt + i * KB1
    pltpu.sync_copy(sd_hbm.at[pl.ds(row0, KB1)], sdb)
    handles = [pltpu.async_copy(x_hbm.at[sdb.at[b, 0]], rows.at[b], sem)
               for b in range(KB1)]
    pltpu.sync_copy(ea_hbm.at[pl.ds(row0, KB1)], eab)
    for h in handles:
      h.wait()
    sc = [pltpu.async_copy(rows.at[b], accx_s.at[sdb.at[b, 1]], sem2, add=True)
          for b in range(KB1)]
    sc += [pltpu.async_copy(eab.at[b], acce_s.at[sdb.at[b, 1]], sem2, add=True)
           for b in range(KB1)]
    for h in sc:
      h.wait()
    return carry

  lax.fori_loop(0, nb, block, 0)
  plsc.subcore_barrier()
  pltpu.sync_copy(accx_s.at[pl.ds(s * rpt, rpt)],
                  outx_hbm.at[c, pl.ds(s * rpt, rpt)])
  pltpu.sync_copy(acce_s.at[pl.ds(s * rpt, rpt)],
                  oute_hbm.at[c, pl.ds(s * rpt, rpt)])


def _roundT_body(nb, cpt, rpt,
                 w_hbm, sd_hbm, z16_hbm, out_hbm,
                 sdb, rows, acc_s, sem, sem2):
  c = lax.axis_index("c")
  s = lax.axis_index("s")
  w = c * NS + s
  pltpu.sync_copy(z16_hbm, acc_s.at[pl.ds(s * rpt, rpt)])
  plsc.subcore_barrier()

  def block(i, carry):
    row0 = w * cpt + i * KBT
    pltpu.sync_copy(sd_hbm.at[pl.ds(row0, KBT)], sdb)
    handles = [pltpu.async_copy(w_hbm.at[sdb.at[b, 0]], rows.at[b], sem)
               for b in range(KBT)]
    for h in handles:
      h.wait()
    sc = [pltpu.async_copy(rows.at[b], acc_s.at[sdb.at[b, 1]], sem2, add=True)
          for b in range(KBT)]
    for h in sc:
      h.wait()
    return carry

  lax.fori_loop(0, nb, block, 0)
  plsc.subcore_barrier()
  pltpu.sync_copy(acc_s.at[pl.ds(s * rpt, rpt)],
                  out_hbm.at[c, pl.ds(s * rpt, rpt)])


@functools.lru_cache(maxsize=None)
def _make_kernels(n, e, d, dep):
  cpt = -(-e // (NW * CHUNK))
  blk = math.lcm(KB1, KBT)
  cpt = -(-cpt // blk) * blk
  e_pad = NW * cpt * CHUNK
  rpt = -(-(n + 1) // (NS * 8)) * 8
  n_pad = NS * rpt

  mesh = plsc.VectorSubcoreMesh(core_axis_name="c", subcore_axis_name="s")
  f32 = jnp.float32
  params = pltpu.CompilerParams(use_tc_tiling_on_sc=False)

  k1 = pl.kernel(
      functools.partial(_round1_body, cpt // KB1, cpt, rpt),
      out_type=(jax.ShapeDtypeStruct((NC, n_pad, d), f32),
                jax.ShapeDtypeStruct((NC, n_pad, dep), f32)),
      mesh=mesh,
      scratch_types=[
          pltpu.VMEM((KB1, 2, CHUNK), jnp.int32),
          pltpu.VMEM((KB1, CHUNK, d), f32),
          pltpu.VMEM((KB1, CHUNK, dep), f32),
          pltpu.VMEM_SHARED((n_pad, d), f32),
          pltpu.VMEM_SHARED((n_pad, dep), f32),
          pltpu.SemaphoreType.DMA,
          pltpu.SemaphoreType.DMA,
      ],
      compiler_params=params,
      name="mpnn_round1",
  )

  dw = d + dep
  kT = pl.kernel(
      functools.partial(_roundT_body, cpt // KBT, cpt, rpt),
      out_type=jax.ShapeDtypeStruct((NC, n_pad, dw), f32),
      mesh=mesh,
      scratch_types=[
          pltpu.VMEM((KBT, 2, CHUNK), jnp.int32),
          pltpu.VMEM((KBT, CHUNK, dw), f32),
          pltpu.VMEM_SHARED((n_pad, dw), f32),
          pltpu.SemaphoreType.DMA,
          pltpu.SemaphoreType.DMA,
      ],
      compiler_params=params,
      name="mpnn_roundT",
  )
  return k1, kT, cpt, e_pad, rpt, n_pad


def kernel(x, edge_index, edge_attr):
  n, d = x.shape
  e, de = edge_attr.shape
  dep = -(-de // 8) * 8
  k1, kT, cpt, e_pad, rpt, n_pad = _make_kernels(n, e, d, dep)

  src = edge_index[0].astype(jnp.int32)
  dst = edge_index[1].astype(jnp.int32)
  pad_e = e_pad - e
  src_p = jnp.concatenate([src, jnp.full((pad_e,), n, jnp.int32)])
  dst_p = jnp.concatenate([dst, jnp.full((pad_e,), n, jnp.int32)])
  ea_p = jnp.zeros((e_pad, dep), jnp.float32).at[:e, :de].set(edge_attr)
  x_p = jnp.concatenate([x, jnp.zeros((n_pad - n, d), jnp.float32)], axis=0)

  sd2 = jnp.stack([src_p.reshape(NW * cpt, CHUNK),
                   dst_p.reshape(NW * cpt, CHUNK)], axis=1)
  ea3 = ea_p.reshape(NW * cpt, CHUNK, dep)

  z8 = jnp.zeros((rpt, d), jnp.float32)
  z16 = jnp.zeros((rpt, d + dep), jnp.float32)

  px, pe = k1(x_p, sd2, ea3, z8)
  w1 = jnp.concatenate([px[0] + px[1], pe[0] + pe[1]], axis=1)
  p2 = kT(w1, sd2, z16)
  w2 = p2[0] + p2[1]
  p3 = kT(w2, sd2, z16)
  w3 = p3[0] + p3[1]

  a1, b1 = w1[:n, :d], w1[:n, d:d + de]
  a2, b2 = w2[:n, :d], w2[:n, d:d + de]
  a3, b3 = w3[:n, :d], w3[:n, d:d + de]
  return jnp.concatenate(
      [x[:, :d], a1, b1,
       a1, a2, b2, b1,
       a1, a2, b2, a2, a3, b3, b2, b1], axis=1)


# speedup vs baseline: 1.2288x; 1.2288x over previous
import functools
import math

import jax
import jax.numpy as jnp
from jax import lax
from jax.experimental import pallas as pl
from jax.experimental.pallas import tpu as pltpu, tpu_sc as plsc

NC = 2
NS = 16
NW = NC * NS
CHUNK = 128
KB1 = 8
KBT = 8


def _round1_body(nb, cpt, rpt,
                 x_hbm, sd_hbm, ea_hbm, z8_hbm,
                 outx_hbm, oute_hbm,
                 sdb, rows, eab, accx_s, acce_s, sem, sem2):
  c = lax.axis_index("c")
  s = lax.axis_index("s")
  w = c * NS + s
  pltpu.sync_copy(z8_hbm, accx_s.at[pl.ds(s * rpt, rpt)])
  pltpu.sync_copy(z8_hbm, acce_s.at[pl.ds(s * rpt, rpt)])
  plsc.subcore_barrier()

  def block(i, carry):
    row0 = w * cpt + i * KB1
    pltpu.sync_copy(sd_hbm.at[pl.ds(row0, KB1)], sdb)
    handles = [pltpu.async_copy(x_hbm.at[sdb.at[b, 0]], rows.at[b], sem)
               for b in range(KB1)]
    pltpu.sync_copy(ea_hbm.at[pl.ds(row0, KB1)], eab)
    for h in handles:
      h.wait()
    sc = [pltpu.async_copy(rows.at[b], accx_s.at[sdb.at[b, 1]], sem2, add=True)
          for b in range(KB1)]
    sc += [pltpu.async_copy(eab.at[b], acce_s.at[sdb.at[b, 1]], sem2, add=True)
           for b in range(KB1)]
    for h in sc:
      h.wait()
    return carry

  lax.fori_loop(0, nb, block, 0)
  plsc.subcore_barrier()
  pltpu.sync_copy(accx_s.at[pl.ds(s * rpt, rpt)],
                  outx_hbm.at[c, pl.ds(s * rpt, rpt)])
  pltpu.sync_copy(acce_s.at[pl.ds(s * rpt, rpt)],
                  oute_hbm.at[c, pl.ds(s * rpt, rpt)])


def _roundT_body(nb, cpt, rpt,
                 w_hbm, sd_hbm, z16_hbm, out_hbm,
                 sdb, rows, acc_s, sem, sem2):
  c = lax.axis_index("c")
  s = lax.axis_index("s")
  w = c * NS + s
  pltpu.sync_copy(z16_hbm, acc_s.at[pl.ds(s * rpt, rpt)])
  plsc.subcore_barrier()

  def block(i, carry):
    row0 = w * cpt + i * KBT
    pltpu.sync_copy(sd_hbm.at[pl.ds(row0, KBT)], sdb)
    handles = [pltpu.async_copy(w_hbm.at[sdb.at[b, 0]], rows.at[b], sem)
               for b in range(KBT)]
    for h in handles:
      h.wait()
    sc = [pltpu.async_copy(rows.at[b], acc_s.at[sdb.at[b, 1]], sem2, add=True)
          for b in range(KBT)]
    for h in sc:
      h.wait()
    return carry

  lax.fori_loop(0, nb, block, 0)
  plsc.subcore_barrier()
  pltpu.sync_copy(acc_s.at[pl.ds(s * rpt, rpt)],
                  out_hbm.at[c, pl.ds(s * rpt, rpt)])


@functools.lru_cache(maxsize=None)
def _make_kernels(n, e, d, dep):
  cpt = -(-e // (NW * CHUNK))
  blk = math.lcm(KB1, KBT)
  cpt = -(-cpt // blk) * blk
  e_pad = NW * cpt * CHUNK
  rpt = -(-(n + 1) // (NS * 8)) * 8
  n_pad = NS * rpt

  mesh = plsc.VectorSubcoreMesh(core_axis_name="c", subcore_axis_name="s")
  f32 = jnp.float32
  params = pltpu.CompilerParams(use_tc_tiling_on_sc=False)

  k1 = pl.kernel(
      functools.partial(_round1_body, cpt // KB1, cpt, rpt),
      out_type=(jax.ShapeDtypeStruct((NC, n_pad, d), f32),
                jax.ShapeDtypeStruct((NC, n_pad, dep), f32)),
      mesh=mesh,
      scratch_types=[
          pltpu.VMEM((KB1, 2, CHUNK), jnp.int32),
          pltpu.VMEM((KB1, CHUNK, d), f32),
          pltpu.VMEM((KB1, CHUNK, dep), f32),
          pltpu.VMEM_SHARED((n_pad, d), f32),
          pltpu.VMEM_SHARED((n_pad, dep), f32),
          pltpu.SemaphoreType.DMA,
          pltpu.SemaphoreType.DMA,
      ],
      compiler_params=params,
      name="mpnn_round1",
  )

  dw = d + dep
  kT = pl.kernel(
      functools.partial(_roundT_body, cpt // KBT, cpt, rpt),
      out_type=jax.ShapeDtypeStruct((NC, n_pad, dw), f32),
      mesh=mesh,
      scratch_types=[
          pltpu.VMEM((KBT, 2, CHUNK), jnp.int32),
          pltpu.VMEM((KBT, CHUNK, dw), f32),
          pltpu.VMEM_SHARED((n_pad, dw), f32),
          pltpu.SemaphoreType.DMA,
          pltpu.SemaphoreType.DMA,
      ],
      compiler_params=params,
      name="mpnn_roundT",
  )
  return k1, kT, cpt, e_pad, rpt, n_pad


def kernel(x, edge_index, edge_attr):
  n, d = x.shape
  e, de = edge_attr.shape
  dep = -(-de // 8) * 8
  k1, kT, cpt, e_pad, rpt, n_pad = _make_kernels(n, e, d, dep)

  src = edge_index[0].astype(jnp.int32)
  dst = edge_index[1].astype(jnp.int32)
  pad_e = e_pad - e
  src_p = jnp.concatenate([src, jnp.full((pad_e,), n, jnp.int32)])
  dst_p = jnp.concatenate([dst, jnp.full((pad_e,), n, jnp.int32)])
  ea_p = jnp.zeros((e_pad, dep), jnp.float32).at[:e, :de].set(edge_attr)
  x_p = jnp.concatenate([x, jnp.zeros((n_pad - n, d), jnp.float32)], axis=0)

  sd2 = jnp.stack([src_p.reshape(NW * cpt, CHUNK),
                   dst_p.reshape(NW * cpt, CHUNK)], axis=1)
  ea3 = ea_p.reshape(NW * cpt, CHUNK, dep)

  z8 = jnp.zeros((rpt, d), jnp.float32)
  z16 = jnp.zeros((rpt, d + dep), jnp.float32)

  px, pe = k1(x_p, sd2, ea3, z8)
  w1 = jnp.concatenate([px[0] + px[1], pe[0] + pe[1]], axis=1)
  p2 = kT(w1, sd2, z16)
  w2 = p2[0] + p2[1]
  p3 = kT(w2, sd2, z16)
  w3 = p3[0] + p3[1]

  a1, b1 = w1[:n, :d], w1[:n, d:d + de]
  a2, b2 = w2[:n, :d], w2[:n, d:d + de]
  a3, b3 = w3[:n, :d], w3[:n, d:d + de]
  return jnp.concatenate(
      [x[:, :d], a1, b1,
       a1, a2, b2, b1,
       a1, a2, b2, a2, a3, b3, b2, b1], axis=1)


# speedup vs baseline: 1.2473x; 1.0150x over previous
import functools
import math

import jax
import jax.numpy as jnp
from jax import lax
from jax.experimental import pallas as pl
from jax.experimental.pallas import tpu as pltpu, tpu_sc as plsc

NC = 2
NS = 16
NW = NC * NS
CHUNK = 128
KB1 = 8
KBT = 8


def _round1_body(nb, cpt, rpt,
                 x_hbm, sd_hbm, ea_hbm, z8_hbm,
                 outx_hbm, oute_hbm,
                 sdb, rows, eab, accx_s, acce_s, sem, sem2):
  c = lax.axis_index("c")
  s = lax.axis_index("s")
  w = c * NS + s
  pltpu.sync_copy(z8_hbm, accx_s.at[pl.ds(s * rpt, rpt)])
  pltpu.sync_copy(z8_hbm, acce_s.at[pl.ds(s * rpt, rpt)])
  plsc.subcore_barrier()

  def block(i, carry):
    row0 = w * cpt + i * KB1
    pltpu.sync_copy(sd_hbm.at[pl.ds(row0, KB1)], sdb)
    handles = [pltpu.async_copy(x_hbm.at[sdb.at[b, 0]], rows.at[b], sem)
               for b in range(KB1)]
    pltpu.sync_copy(ea_hbm.at[pl.ds(row0, KB1)], eab)
    sc = [pltpu.async_copy(eab.at[b], acce_s.at[sdb.at[b, 1]], sem2, add=True)
          for b in range(KB1)]
    for b in range(KB1):
      handles[b].wait()
      sc.append(pltpu.async_copy(rows.at[b], accx_s.at[sdb.at[b, 1]],
                                 sem2, add=True))
    for h in sc:
      h.wait()
    return carry

  lax.fori_loop(0, nb, block, 0)
  plsc.subcore_barrier()
  pltpu.sync_copy(accx_s.at[pl.ds(s * rpt, rpt)],
                  outx_hbm.at[c, pl.ds(s * rpt, rpt)])
  pltpu.sync_copy(acce_s.at[pl.ds(s * rpt, rpt)],
                  oute_hbm.at[c, pl.ds(s * rpt, rpt)])


def _roundT_body(nb, cpt, rpt,
                 w_hbm, sd_hbm, z16_hbm, out_hbm,
                 sdb, rows, acc_s, sem, sem2):
  c = lax.axis_index("c")
  s = lax.axis_index("s")
  w = c * NS + s
  pltpu.sync_copy(z16_hbm, acc_s.at[pl.ds(s * rpt, rpt)])
  plsc.subcore_barrier()

  def block(i, carry):
    row0 = w * cpt + i * KBT
    pltpu.sync_copy(sd_hbm.at[pl.ds(row0, KBT)], sdb)
    handles = [pltpu.async_copy(w_hbm.at[sdb.at[b, 0]], rows.at[b], sem)
               for b in range(KBT)]
    sc = []
    for b in range(KBT):
      handles[b].wait()
      sc.append(pltpu.async_copy(rows.at[b], acc_s.at[sdb.at[b, 1]],
                                 sem2, add=True))
    for h in sc:
      h.wait()
    return carry

  lax.fori_loop(0, nb, block, 0)
  plsc.subcore_barrier()
  pltpu.sync_copy(acc_s.at[pl.ds(s * rpt, rpt)],
                  out_hbm.at[c, pl.ds(s * rpt, rpt)])


@functools.lru_cache(maxsize=None)
def _make_kernels(n, e, d, dep):
  cpt = -(-e // (NW * CHUNK))
  blk = math.lcm(KB1, KBT)
  cpt = -(-cpt // blk) * blk
  e_pad = NW * cpt * CHUNK
  rpt = -(-(n + 1) // (NS * 8)) * 8
  n_pad = NS * rpt

  mesh = plsc.VectorSubcoreMesh(core_axis_name="c", subcore_axis_name="s")
  f32 = jnp.float32
  params = pltpu.CompilerParams(use_tc_tiling_on_sc=False)

  k1 = pl.kernel(
      functools.partial(_round1_body, cpt // KB1, cpt, rpt),
      out_type=(jax.ShapeDtypeStruct((NC, n_pad, d), f32),
                jax.ShapeDtypeStruct((NC, n_pad, dep), f32)),
      mesh=mesh,
      scratch_types=[
          pltpu.VMEM((KB1, 2, CHUNK), jnp.int32),
          pltpu.VMEM((KB1, CHUNK, d), f32),
          pltpu.VMEM((KB1, CHUNK, dep), f32),
          pltpu.VMEM_SHARED((n_pad, d), f32),
          pltpu.VMEM_SHARED((n_pad, dep), f32),
          pltpu.SemaphoreType.DMA,
          pltpu.SemaphoreType.DMA,
      ],
      compiler_params=params,
      name="mpnn_round1",
  )

  dw = d + dep
  kT = pl.kernel(
      functools.partial(_roundT_body, cpt // KBT, cpt, rpt),
      out_type=jax.ShapeDtypeStruct((NC, n_pad, dw), f32),
      mesh=mesh,
      scratch_types=[
          pltpu.VMEM((KBT, 2, CHUNK), jnp.int32),
          pltpu.VMEM((KBT, CHUNK, dw), f32),
          pltpu.VMEM_SHARED((n_pad, dw), f32),
          pltpu.SemaphoreType.DMA,
          pltpu.SemaphoreType.DMA,
      ],
      compiler_params=params,
      name="mpnn_roundT",
  )
  return k1, kT, cpt, e_pad, rpt, n_pad


def kernel(x, edge_index, edge_attr):
  n, d = x.shape
  e, de = edge_attr.shape
  dep = -(-de // 8) * 8
  k1, kT, cpt, e_pad, rpt, n_pad = _make_kernels(n, e, d, dep)

  src = edge_index[0].astype(jnp.int32)
  dst = edge_index[1].astype(jnp.int32)
  pad_e = e_pad - e
  src_p = jnp.concatenate([src, jnp.full((pad_e,), n, jnp.int32)])
  dst_p = jnp.concatenate([dst, jnp.full((pad_e,), n, jnp.int32)])
  ea_p = jnp.zeros((e_pad, dep), jnp.float32).at[:e, :de].set(edge_attr)
  x_p = jnp.concatenate([x, jnp.zeros((n_pad - n, d), jnp.float32)], axis=0)

  sd2 = jnp.stack([src_p.reshape(NW * cpt, CHUNK),
                   dst_p.reshape(NW * cpt, CHUNK)], axis=1)
  ea3 = ea_p.reshape(NW * cpt, CHUNK, dep)

  z8 = jnp.zeros((rpt, d), jnp.float32)
  z16 = jnp.zeros((rpt, d + dep), jnp.float32)

  px, pe = k1(x_p, sd2, ea3, z8)
  w1 = jnp.concatenate([px[0] + px[1], pe[0] + pe[1]], axis=1)
  p2 = kT(w1, sd2, z16)
  w2 = p2[0] + p2[1]
  p3 = kT(w2, sd2, z16)
  w3 = p3[0] + p3[1]

  a1, b1 = w1[:n, :d], w1[:n, d:d + de]
  a2, b2 = w2[:n, :d], w2[:n, d:d + de]
  a3, b3 = w3[:n, :d], w3[:n, d:d + de]
  return jnp.concatenate(
      [x[:, :d], a1, b1,
       a1, a2, b2, b1,
       a1, a2, b2, a2, a3, b3, b2, b1], axis=1)
